# baseline (speedup 1.0000x reference)
import functools

import jax
import jax.numpy as jnp
from jax import lax
from jax.experimental import pallas as pl
from jax.experimental.pallas import tpu as pltpu
from jax.experimental.pallas import tpu_sc as plsc

N = 10000
NP = 10240
E = 320000
NC = 2
NS = 16
NW = NC * NS
EW = E // NW
CH = 80
NCH = EW // CH
PT = NP // NS
W16 = 16

_f32 = jnp.float32


def _mesh():
    return plsc.VectorSubcoreMesh(core_axis_name="c", subcore_axis_name="s")


_SC_PARAMS = pltpu.CompilerParams(use_tc_tiling_on_sc=False)


def _sc_degrees(src_r, dst_r, ones16, zeros16):
    @functools.partial(
        pl.kernel,
        out_type=(
            jax.ShapeDtypeStruct((NC, NP, W16), _f32),
            jax.ShapeDtypeStruct((NC, NP, W16), _f32),
        ),
        mesh=_mesh(),
        compiler_params=_SC_PARAMS,
        scratch_types=[
            pltpu.VMEM((NCH, CH), jnp.int32),
            pltpu.VMEM((NCH, CH), jnp.int32),
            pltpu.VMEM((CH, W16), _f32),
            pltpu.VMEM((CH, W16), _f32),
            pltpu.VMEM_SHARED((NP, W16), _f32),
            pltpu.VMEM_SHARED((NP, W16), _f32),
            pltpu.SemaphoreType.DMA,
        ],
    )
    def deg_kernel(src_hbm, dst_hbm, ones_hbm, z16_hbm, dego_hbm, degi_hbm,
                   src_v, dst_v, ones_v, obuf, dsh_o, dsh_i, sem):
        cid = lax.axis_index("c")
        sid = lax.axis_index("s")
        wid = sid * NC + cid
        pltpu.sync_copy(src_hbm.at[wid], src_v)
        pltpu.sync_copy(dst_hbm.at[wid], dst_v)
        pltpu.sync_copy(ones_hbm, ones_v)
        pltpu.sync_copy(z16_hbm, dsh_o.at[pl.ds(sid * PT, PT)])
        pltpu.sync_copy(z16_hbm, dsh_i.at[pl.ds(sid * PT, PT)])
        plsc.subcore_barrier()

        def group(i, carry):
            ds = []
            for b in range(4):
                j = i * 4 + b
                ds.append(pltpu.async_copy(
                    ones_v, dsh_o.at[src_v.at[j]], sem, add=True))
                ds.append(pltpu.async_copy(
                    ones_v, dsh_i.at[dst_v.at[j]], sem, add=True))
            for dsc in ds:
                dsc.wait()
            return carry

        lax.fori_loop(0, NCH // 4, group, 0)
        for j in range(NCH - NCH % 4, NCH):
            pltpu.sync_copy(ones_v, dsh_o.at[src_v.at[j]], add=True)
            pltpu.sync_copy(ones_v, dsh_i.at[dst_v.at[j]], add=True)
        plsc.subcore_barrier()

        def out_body(k, carry):
            row = sid * PT + k * CH
            pltpu.sync_copy(dsh_o.at[pl.ds(row, CH)], obuf)
            pltpu.sync_copy(obuf, dego_hbm.at[cid].at[pl.ds(row, CH)])
            pltpu.sync_copy(dsh_i.at[pl.ds(row, CH)], obuf)
            pltpu.sync_copy(obuf, degi_hbm.at[cid].at[pl.ds(row, CH)])
            return carry

        lax.fori_loop(0, PT // CH, out_body, 0)

    return deg_kernel(src_r, dst_r, ones16, zeros16)


PH = 63


def _sc_propagate(src_r, dst_r, u, din16, zeros_tile, zeros16):
    @functools.partial(
        pl.kernel,
        out_type=(
            jax.ShapeDtypeStruct((NC, NP, 128), _f32),
            jax.ShapeDtypeStruct((NC, NP, W16), _f32),
        ),
        mesh=_mesh(),
        compiler_params=_SC_PARAMS,
        scratch_types=[
            pltpu.VMEM((PH, CH), jnp.int32),
            pltpu.VMEM((PH, CH), jnp.int32),
            pltpu.VMEM((CH, 128), _f32),
            pltpu.VMEM((CH, 128), _f32),
            pltpu.VMEM((CH, W16), _f32),
            pltpu.VMEM((CH, W16), _f32),
            pltpu.VMEM_SHARED((NP, 128), _f32),
            pltpu.VMEM_SHARED((NP, W16), _f32),
            pltpu.SemaphoreType.DMA,
            pltpu.SemaphoreType.DMA,
            pltpu.SemaphoreType.DMA,
            pltpu.SemaphoreType.DMA,
            pltpu.SemaphoreType.DMA,
            pltpu.SemaphoreType.DMA,
        ],
    )
    def prop_kernel(src_hbm, dst_hbm, u_hbm, din_hbm, zt_hbm, z16_hbm,
                    agg_out, c_out,
                    src_v, dst_v, gbuf0, gbuf1, dbuf0, dbuf1, agg_sh, c_sh,
                    gsem0, gsem1, dsem0, dsem1, ssem, csem):
        cid = lax.axis_index("c")
        sid = lax.axis_index("s")
        wid = sid * NC + cid
        pltpu.sync_copy(zt_hbm, agg_sh.at[pl.ds(sid * PT, PT)])
        pltpu.sync_copy(z16_hbm, c_sh.at[pl.ds(sid * PT, PT)])
        plsc.subcore_barrier()

        def g_start(j, buf, sem):
            pltpu.async_copy(u_hbm.at[src_v.at[j]], buf, sem)

        def d_start(j, buf, sem):
            pltpu.async_copy(din_hbm.at[dst_v.at[j]], buf, sem)

        def g_wait(buf, sem):
            pltpu.make_async_copy(u_hbm.at[pl.ds(0, CH)], buf, sem).wait()

        def d_wait(buf, sem):
            pltpu.make_async_copy(din_hbm.at[pl.ds(0, CH)], buf, sem).wait()

        def s_add(j, buf):
            pltpu.sync_copy(buf, agg_sh.at[dst_v.at[j]], add=True)

        def c_add(j, buf):
            pltpu.sync_copy(buf, c_sh.at[src_v.at[j]], add=True)

        for p, n_chunks in enumerate((PH, NCH - PH)):
            base = p * PH
            pltpu.sync_copy(src_hbm.at[wid].at[pl.ds(base, n_chunks)],
                            src_v.at[pl.ds(0, n_chunks)])
            pltpu.sync_copy(dst_hbm.at[wid].at[pl.ds(base, n_chunks)],
                            dst_v.at[pl.ds(0, n_chunks)])
            n_pipe = n_chunks if n_chunks % 2 == 1 else n_chunks - 1
            g_start(0, gbuf0, gsem0)
            d_start(0, dbuf0, dsem0)

            def pair(i, carry):
                j0 = 2 * i
                g_start(j0 + 1, gbuf1, gsem1)
                d_start(j0 + 1, dbuf1, dsem1)
                g_wait(gbuf0, gsem0)
                d_wait(dbuf0, dsem0)
                sa = pltpu.async_copy(gbuf0, agg_sh.at[dst_v.at[j0]], ssem,
                                      add=True)
                ca = pltpu.async_copy(dbuf0, c_sh.at[src_v.at[j0]], csem,
                                      add=True)
                g_wait(gbuf1, gsem1)
                d_wait(dbuf1, dsem1)
                sb = pltpu.async_copy(gbuf1, agg_sh.at[dst_v.at[j0 + 1]],
                                      ssem, add=True)
                cb = pltpu.async_copy(dbuf1, c_sh.at[src_v.at[j0 + 1]],
                                      csem, add=True)
                sa.wait()
                ca.wait()
                g_start(j0 + 2, gbuf0, gsem0)
                d_start(j0 + 2, dbuf0, dsem0)
                sb.wait()
                cb.wait()
                return carry

            lax.fori_loop(0, (n_pipe - 1) // 2, pair, 0)
            g_wait(gbuf0, gsem0)
            d_wait(dbuf0, dsem0)
            s_add(n_pipe - 1, gbuf0)
            c_add(n_pipe - 1, dbuf0)
            if n_chunks % 2 == 0:
                pltpu.sync_copy(u_hbm.at[src_v.at[n_chunks - 1]], gbuf1)
                pltpu.sync_copy(din_hbm.at[dst_v.at[n_chunks - 1]], dbuf1)
                s_add(n_chunks - 1, gbuf1)
                c_add(n_chunks - 1, dbuf1)
        plsc.subcore_barrier()

        def out_body(k, carry):
            row = sid * PT + k * CH
            pltpu.sync_copy(agg_sh.at[pl.ds(row, CH)], gbuf0)
            pltpu.sync_copy(gbuf0, agg_out.at[cid].at[pl.ds(row, CH)])
            pltpu.sync_copy(c_sh.at[pl.ds(row, CH)], dbuf0)
            pltpu.sync_copy(dbuf0, c_out.at[cid].at[pl.ds(row, CH)])
            return carry

        lax.fori_loop(0, PT // CH, out_body, 0)

    return prop_kernel(src_r, dst_r, u, din16, zeros_tile, zeros16)


def _tc_pre_body(x_ref, w_ref, dpo_ref, dpi_ref, u_ref, din_ref, dout_ref):
    do = (dpo_ref[0] + dpo_ref[1])[:, :1]
    di = (dpi_ref[0] + dpi_ref[1])[:, :1]
    dout = lax.rsqrt(jnp.maximum(do, 1.0))
    din = lax.rsqrt(jnp.maximum(di, 1.0))
    dout_ref[...] = dout
    din_ref[...] = jnp.broadcast_to(din, (NP, W16))
    u_ref[...] = jnp.dot(x_ref[...], w_ref[...],
                         preferred_element_type=jnp.float32) * dout[:N]


def _tc_pre(x, W1, degp_out, degp_in):
    return pl.pallas_call(
        _tc_pre_body,
        out_shape=(
            jax.ShapeDtypeStruct((N, 128), _f32),
            jax.ShapeDtypeStruct((NP, W16), _f32),
            jax.ShapeDtypeStruct((NP, 1), _f32),
        ),
    )(x, W1, degp_out, degp_in)


def _tc_post_body(aggp_ref, cp_ref, din_ref, dout_ref, b1_ref, w2_ref,
                  b2_ref, o_ref):
    agg = aggp_ref[0] + aggp_ref[1]
    din = din_ref[...][:, :1]
    h = jnp.maximum(agg * din + b1_ref[...][None, :], 0.0)
    z = h * dout_ref[...]
    c = (cp_ref[0] + cp_ref[1])[:, :1]
    v = jnp.sum(z * c, axis=0, keepdims=True)
    o_ref[...] = (jnp.dot(v * (1.0 / N), w2_ref[...],
                          preferred_element_type=jnp.float32)
                  + b2_ref[...][None, :])


def _tc_post(aggp, cp, din16, dout_col, b1, W2, b2):
    return pl.pallas_call(
        _tc_post_body,
        out_shape=jax.ShapeDtypeStruct((1, 64), _f32),
    )(aggp, cp, din16, dout_col, b1, W2, b2)


def kernel(x, edge_index, e_h, W1, b1, W2, b2, param_mu, param_sigma):
    del e_h, param_mu, param_sigma
    src_s = edge_index[0].reshape(NW, NCH, CH)
    dst_s = edge_index[1].reshape(NW, NCH, CH)
    ones16 = jnp.ones((CH, W16), _f32)
    zeros16 = jnp.zeros((PT, W16), _f32)
    zeros_tile = jnp.zeros((PT, 128), _f32)

    degp_out, degp_in = _sc_degrees(src_s, dst_s, ones16, zeros16)
    u, din16, dout_col = _tc_pre(x, W1, degp_out, degp_in)
    aggp, cp = _sc_propagate(src_s, dst_s, u, din16, zeros_tile, zeros16)
    return _tc_post(aggp, cp, din16, dout_col, b1, W2, b2)


# speedup vs baseline: 1.0224x; 1.0224x over previous
import functools

import jax
import jax.numpy as jnp
from jax import lax
from jax.experimental import pallas as pl
from jax.experimental.pallas import tpu as pltpu
from jax.experimental.pallas import tpu_sc as plsc

N = 10000
NP = 10240
E = 320000
NC = 2
NS = 16
NW = NC * NS
EW = E // NW
CH = 80
NCH = EW // CH
PT = NP // NS
W16 = 16

_f32 = jnp.float32


def _mesh():
    return plsc.VectorSubcoreMesh(core_axis_name="c", subcore_axis_name="s")


_SC_PARAMS = pltpu.CompilerParams(use_tc_tiling_on_sc=False)


def _sc_degrees(src_r, dst_r, ones16, zeros16):
    @functools.partial(
        pl.kernel,
        out_type=(
            jax.ShapeDtypeStruct((NC, NP, W16), _f32),
            jax.ShapeDtypeStruct((NC, NP, W16), _f32),
        ),
        mesh=_mesh(),
        compiler_params=_SC_PARAMS,
        scratch_types=[
            pltpu.VMEM((NCH, CH), jnp.int32),
            pltpu.VMEM((NCH, CH), jnp.int32),
            pltpu.VMEM((CH, W16), _f32),
            pltpu.VMEM((CH, W16), _f32),
            pltpu.VMEM_SHARED((NP, W16), _f32),
            pltpu.VMEM_SHARED((NP, W16), _f32),
            pltpu.SemaphoreType.DMA,
        ],
    )
    def deg_kernel(src_hbm, dst_hbm, ones_hbm, z16_hbm, dego_hbm, degi_hbm,
                   src_v, dst_v, ones_v, obuf, dsh_o, dsh_i, sem):
        cid = lax.axis_index("c")
        sid = lax.axis_index("s")
        wid = sid * NC + cid
        pltpu.sync_copy(src_hbm.at[wid], src_v)
        pltpu.sync_copy(dst_hbm.at[wid], dst_v)
        pltpu.sync_copy(ones_hbm, ones_v)
        pltpu.sync_copy(z16_hbm, dsh_o.at[pl.ds(sid * PT, PT)])
        pltpu.sync_copy(z16_hbm, dsh_i.at[pl.ds(sid * PT, PT)])
        plsc.subcore_barrier()

        def group(i, carry):
            ds = []
            for b in range(4):
                j = i * 4 + b
                ds.append(pltpu.async_copy(
                    ones_v, dsh_o.at[src_v.at[j]], sem, add=True))
                ds.append(pltpu.async_copy(
                    ones_v, dsh_i.at[dst_v.at[j]], sem, add=True))
            for dsc in ds:
                dsc.wait()
            return carry

        lax.fori_loop(0, NCH // 4, group, 0)
        for j in range(NCH - NCH % 4, NCH):
            pltpu.sync_copy(ones_v, dsh_o.at[src_v.at[j]], add=True)
            pltpu.sync_copy(ones_v, dsh_i.at[dst_v.at[j]], add=True)
        plsc.subcore_barrier()

        def out_body(k, carry):
            row = sid * PT + k * CH
            pltpu.sync_copy(dsh_o.at[pl.ds(row, CH)], obuf)
            pltpu.sync_copy(obuf, dego_hbm.at[cid].at[pl.ds(row, CH)])
            pltpu.sync_copy(dsh_i.at[pl.ds(row, CH)], obuf)
            pltpu.sync_copy(obuf, degi_hbm.at[cid].at[pl.ds(row, CH)])
            return carry

        lax.fori_loop(0, PT // CH, out_body, 0)

    return deg_kernel(src_r, dst_r, ones16, zeros16)


PH = 63


def _sc_propagate(src_r, dst_r, u, din16, zeros_tile, zeros16):
    @functools.partial(
        pl.kernel,
        out_type=(
            jax.ShapeDtypeStruct((NC, NP, 128), _f32),
            jax.ShapeDtypeStruct((NC, NP, W16), _f32),
        ),
        mesh=_mesh(),
        compiler_params=_SC_PARAMS,
        scratch_types=[
            pltpu.VMEM((PH, CH), jnp.int32),
            pltpu.VMEM((PH, CH), jnp.int32),
            pltpu.VMEM((CH, 128), _f32),
            pltpu.VMEM((CH, 128), _f32),
            pltpu.VMEM((CH, W16), _f32),
            pltpu.VMEM((CH, W16), _f32),
            pltpu.VMEM_SHARED((NP, 128), _f32),
            pltpu.VMEM_SHARED((NP, W16), _f32),
            pltpu.SemaphoreType.DMA,
            pltpu.SemaphoreType.DMA,
            pltpu.SemaphoreType.DMA,
            pltpu.SemaphoreType.DMA,
        ],
    )
    def prop_kernel(src_hbm, dst_hbm, u_hbm, din_hbm, zt_hbm, z16_hbm,
                    agg_out, c_out,
                    src_v, dst_v, gbuf0, gbuf1, dbuf0, dbuf1, agg_sh, c_sh,
                    gsem0, gsem1, dsem0, dsem1):
        cid = lax.axis_index("c")
        sid = lax.axis_index("s")
        wid = sid * NC + cid
        pltpu.sync_copy(zt_hbm, agg_sh.at[pl.ds(sid * PT, PT)])
        pltpu.sync_copy(z16_hbm, c_sh.at[pl.ds(sid * PT, PT)])
        plsc.subcore_barrier()

        def g_start(j, buf, sem):
            pltpu.async_copy(u_hbm.at[src_v.at[j]], buf, sem)

        def d_start(j, buf, sem):
            pltpu.async_copy(din_hbm.at[dst_v.at[j]], buf, sem)

        def g_wait(buf, sem):
            pltpu.make_async_copy(u_hbm.at[pl.ds(0, CH)], buf, sem).wait()

        def d_wait(buf, sem):
            pltpu.make_async_copy(din_hbm.at[pl.ds(0, CH)], buf, sem).wait()

        def s_add(j, buf):
            pltpu.sync_copy(buf, agg_sh.at[dst_v.at[j]], add=True)

        def c_add(j, buf):
            pltpu.sync_copy(buf, c_sh.at[src_v.at[j]], add=True)

        for p, n_chunks in enumerate((PH, NCH - PH)):
            base = p * PH
            pltpu.sync_copy(src_hbm.at[wid].at[pl.ds(base, n_chunks)],
                            src_v.at[pl.ds(0, n_chunks)])
            pltpu.sync_copy(dst_hbm.at[wid].at[pl.ds(base, n_chunks)],
                            dst_v.at[pl.ds(0, n_chunks)])
            n_pipe = n_chunks if n_chunks % 2 == 1 else n_chunks - 1
            g_start(0, gbuf0, gsem0)
            d_start(0, dbuf0, dsem0)

            def pair(i, carry):
                j0 = 2 * i
                g_start(j0 + 1, gbuf1, gsem1)
                d_start(j0 + 1, dbuf1, dsem1)
                g_wait(gbuf0, gsem0)
                d_wait(dbuf0, dsem0)
                s_add(j0, gbuf0)
                c_add(j0, dbuf0)
                g_start(j0 + 2, gbuf0, gsem0)
                d_start(j0 + 2, dbuf0, dsem0)
                g_wait(gbuf1, gsem1)
                d_wait(dbuf1, dsem1)
                s_add(j0 + 1, gbuf1)
                c_add(j0 + 1, dbuf1)
                return carry

            lax.fori_loop(0, (n_pipe - 1) // 2, pair, 0)
            g_wait(gbuf0, gsem0)
            d_wait(dbuf0, dsem0)
            s_add(n_pipe - 1, gbuf0)
            c_add(n_pipe - 1, dbuf0)
            if n_chunks % 2 == 0:
                pltpu.sync_copy(u_hbm.at[src_v.at[n_chunks - 1]], gbuf1)
                pltpu.sync_copy(din_hbm.at[dst_v.at[n_chunks - 1]], dbuf1)
                s_add(n_chunks - 1, gbuf1)
                c_add(n_chunks - 1, dbuf1)
        plsc.subcore_barrier()

        def out_body(k, carry):
            row = sid * PT + k * CH
            pltpu.sync_copy(agg_sh.at[pl.ds(row, CH)], gbuf0)
            pltpu.sync_copy(gbuf0, agg_out.at[cid].at[pl.ds(row, CH)])
            pltpu.sync_copy(c_sh.at[pl.ds(row, CH)], dbuf0)
            pltpu.sync_copy(dbuf0, c_out.at[cid].at[pl.ds(row, CH)])
            return carry

        lax.fori_loop(0, PT // CH, out_body, 0)

    return prop_kernel(src_r, dst_r, u, din16, zeros_tile, zeros16)


def _tc_mm_body(x_ref, w_ref, u0_ref):
    u0_ref[...] = jnp.dot(x_ref[...], w_ref[...],
                          preferred_element_type=jnp.float32)


def _tc_mm(x, W1):
    return pl.pallas_call(
        _tc_mm_body,
        out_shape=jax.ShapeDtypeStruct((N, 128), _f32),
    )(x, W1)


def _tc_pre_body(u0_ref, dpo_ref, dpi_ref, u_ref, din_ref, dout_ref):
    do = (dpo_ref[0] + dpo_ref[1])[:, :1]
    di = (dpi_ref[0] + dpi_ref[1])[:, :1]
    dout = lax.rsqrt(jnp.maximum(do, 1.0))
    din = lax.rsqrt(jnp.maximum(di, 1.0))
    dout_ref[...] = dout
    din_ref[...] = jnp.broadcast_to(din, (NP, W16))
    u_ref[...] = u0_ref[...] * dout[:N]


def _tc_pre(u0, degp_out, degp_in):
    return pl.pallas_call(
        _tc_pre_body,
        out_shape=(
            jax.ShapeDtypeStruct((N, 128), _f32),
            jax.ShapeDtypeStruct((NP, W16), _f32),
            jax.ShapeDtypeStruct((NP, 1), _f32),
        ),
    )(u0, degp_out, degp_in)


def _tc_post_body(aggp_ref, cp_ref, din_ref, dout_ref, b1_ref, w2_ref,
                  b2_ref, o_ref):
    agg = aggp_ref[0] + aggp_ref[1]
    din = din_ref[...][:, :1]
    h = jnp.maximum(agg * din + b1_ref[...][None, :], 0.0)
    z = h * dout_ref[...]
    c = (cp_ref[0] + cp_ref[1])[:, :1]
    v = jnp.sum(z * c, axis=0, keepdims=True)
    o_ref[...] = (jnp.dot(v * (1.0 / N), w2_ref[...],
                          preferred_element_type=jnp.float32)
                  + b2_ref[...][None, :])


def _tc_post(aggp, cp, din16, dout_col, b1, W2, b2):
    return pl.pallas_call(
        _tc_post_body,
        out_shape=jax.ShapeDtypeStruct((1, 64), _f32),
    )(aggp, cp, din16, dout_col, b1, W2, b2)


def kernel(x, edge_index, e_h, W1, b1, W2, b2, param_mu, param_sigma):
    del e_h, param_mu, param_sigma
    src_s = edge_index[0].reshape(NW, NCH, CH)
    dst_s = edge_index[1].reshape(NW, NCH, CH)
    ones16 = jnp.ones((CH, W16), _f32)
    zeros16 = jnp.zeros((PT, W16), _f32)
    zeros_tile = jnp.zeros((PT, 128), _f32)

    u0 = _tc_mm(x, W1)
    degp_out, degp_in = _sc_degrees(src_s, dst_s, ones16, zeros16)
    u, din16, dout_col = _tc_pre(u0, degp_out, degp_in)
    aggp, cp = _sc_propagate(src_s, dst_s, u, din16, zeros_tile, zeros16)
    return _tc_post(aggp, cp, din16, dout_col, b1, W2, b2)


# speedup vs baseline: 1.0408x; 1.0180x over previous
import functools

import jax
import jax.numpy as jnp
from jax import lax
from jax.experimental import pallas as pl
from jax.experimental.pallas import tpu as pltpu
from jax.experimental.pallas import tpu_sc as plsc

N = 10000
NP = 10240
E = 320000
NC = 2
NS = 16
NW = NC * NS
EW = E // NW
CH = 80
NCH = EW // CH
PT = NP // NS
W16 = 16

_f32 = jnp.float32


def _mesh():
    return plsc.VectorSubcoreMesh(core_axis_name="c", subcore_axis_name="s")


_SC_PARAMS = pltpu.CompilerParams(use_tc_tiling_on_sc=False)


def _sc_degrees(src_r, dst_r, ones16, zeros16):
    @functools.partial(
        pl.kernel,
        out_type=(
            jax.ShapeDtypeStruct((NC, NP, W16), _f32),
            jax.ShapeDtypeStruct((NC, NP, W16), _f32),
        ),
        mesh=_mesh(),
        compiler_params=_SC_PARAMS,
        scratch_types=[
            pltpu.VMEM((NCH, CH), jnp.int32),
            pltpu.VMEM((NCH, CH), jnp.int32),
            pltpu.VMEM((CH, W16), _f32),
            pltpu.VMEM((CH, W16), _f32),
            pltpu.VMEM_SHARED((NP, W16), _f32),
            pltpu.VMEM_SHARED((NP, W16), _f32),
            pltpu.SemaphoreType.DMA,
        ],
    )
    def deg_kernel(src_hbm, dst_hbm, ones_hbm, z16_hbm, dego_hbm, degi_hbm,
                   src_v, dst_v, ones_v, obuf, dsh_o, dsh_i, sem):
        cid = lax.axis_index("c")
        sid = lax.axis_index("s")
        wid = sid * NC + cid
        pltpu.sync_copy(src_hbm.at[wid], src_v)
        pltpu.sync_copy(dst_hbm.at[wid], dst_v)
        pltpu.sync_copy(ones_hbm, ones_v)
        pltpu.sync_copy(z16_hbm, dsh_o.at[pl.ds(sid * PT, PT)])
        pltpu.sync_copy(z16_hbm, dsh_i.at[pl.ds(sid * PT, PT)])
        plsc.subcore_barrier()

        def group(i, carry):
            ds = []
            for b in range(4):
                j = i * 4 + b
                ds.append(pltpu.async_copy(
                    ones_v, dsh_o.at[src_v.at[j]], sem, add=True))
                ds.append(pltpu.async_copy(
                    ones_v, dsh_i.at[dst_v.at[j]], sem, add=True))
            for dsc in ds:
                dsc.wait()
            return carry

        lax.fori_loop(0, NCH // 4, group, 0)
        for j in range(NCH - NCH % 4, NCH):
            pltpu.sync_copy(ones_v, dsh_o.at[src_v.at[j]], add=True)
            pltpu.sync_copy(ones_v, dsh_i.at[dst_v.at[j]], add=True)
        plsc.subcore_barrier()

        def out_body(k, carry):
            row = sid * PT + k * CH
            pltpu.sync_copy(dsh_o.at[pl.ds(row, CH)], obuf)
            pltpu.sync_copy(obuf, dego_hbm.at[cid].at[pl.ds(row, CH)])
            pltpu.sync_copy(dsh_i.at[pl.ds(row, CH)], obuf)
            pltpu.sync_copy(obuf, degi_hbm.at[cid].at[pl.ds(row, CH)])
            return carry

        lax.fori_loop(0, PT // CH, out_body, 0)

    return deg_kernel(src_r, dst_r, ones16, zeros16)


PH = 63


def _sc_propagate(src_r, dst_r, u, din16, zeros_tile, zeros16):
    @functools.partial(
        pl.kernel,
        out_type=(
            jax.ShapeDtypeStruct((NC, NP, 128), _f32),
            jax.ShapeDtypeStruct((NC, NP, W16), _f32),
        ),
        mesh=_mesh(),
        compiler_params=_SC_PARAMS,
        scratch_types=[
            pltpu.VMEM((PH, CH), jnp.int32),
            pltpu.VMEM((PH, CH), jnp.int32),
            pltpu.VMEM((CH, 128), _f32),
            pltpu.VMEM((CH, 128), _f32),
            pltpu.VMEM((CH, W16), _f32),
            pltpu.VMEM((CH, W16), _f32),
            pltpu.VMEM_SHARED((NP, 128), _f32),
            pltpu.VMEM_SHARED((NP, W16), _f32),
            pltpu.SemaphoreType.DMA,
            pltpu.SemaphoreType.DMA,
            pltpu.SemaphoreType.DMA,
            pltpu.SemaphoreType.DMA,
        ],
    )
    def prop_kernel(src_hbm, dst_hbm, u_hbm, din_hbm, zt_hbm, z16_hbm,
                    agg_out, c_out,
                    src_v, dst_v, gbuf0, gbuf1, dbuf0, dbuf1, agg_sh, c_sh,
                    gsem0, gsem1, dsem0, dsem1):
        cid = lax.axis_index("c")
        sid = lax.axis_index("s")
        wid = sid * NC + cid
        pltpu.sync_copy(zt_hbm, agg_sh.at[pl.ds(sid * PT, PT)])
        pltpu.sync_copy(z16_hbm, c_sh.at[pl.ds(sid * PT, PT)])
        plsc.subcore_barrier()

        def g_start(j, buf, sem):
            pltpu.async_copy(u_hbm.at[src_v.at[j]], buf, sem)

        def d_start(j, buf, sem):
            pltpu.async_copy(din_hbm.at[dst_v.at[j]], buf, sem)

        def g_wait(buf, sem):
            pltpu.make_async_copy(u_hbm.at[pl.ds(0, CH)], buf, sem).wait()

        def d_wait(buf, sem):
            pltpu.make_async_copy(din_hbm.at[pl.ds(0, CH)], buf, sem).wait()

        def s_add(j, buf):
            pltpu.sync_copy(buf, agg_sh.at[dst_v.at[j]], add=True)

        def c_add(j, buf):
            pltpu.sync_copy(buf, c_sh.at[src_v.at[j]], add=True)

        for p, n_chunks in enumerate((PH, NCH - PH)):
            base = p * PH
            pltpu.sync_copy(src_hbm.at[wid].at[pl.ds(base, n_chunks)],
                            src_v.at[pl.ds(0, n_chunks)])
            pltpu.sync_copy(dst_hbm.at[wid].at[pl.ds(base, n_chunks)],
                            dst_v.at[pl.ds(0, n_chunks)])
            n_pipe = n_chunks if n_chunks % 2 == 1 else n_chunks - 1
            g_start(0, gbuf0, gsem0)
            d_start(0, dbuf0, dsem0)

            def pair(i, carry):
                j0 = 2 * i
                g_start(j0 + 1, gbuf1, gsem1)
                d_start(j0 + 1, dbuf1, dsem1)
                g_wait(gbuf0, gsem0)
                d_wait(dbuf0, dsem0)
                s_add(j0, gbuf0)
                c_add(j0, dbuf0)
                g_start(j0 + 2, gbuf0, gsem0)
                d_start(j0 + 2, dbuf0, dsem0)
                g_wait(gbuf1, gsem1)
                d_wait(dbuf1, dsem1)
                s_add(j0 + 1, gbuf1)
                c_add(j0 + 1, dbuf1)
                return carry

            lax.fori_loop(0, (n_pipe - 1) // 2, pair, 0)
            g_wait(gbuf0, gsem0)
            d_wait(dbuf0, dsem0)
            s_add(n_pipe - 1, gbuf0)
            c_add(n_pipe - 1, dbuf0)
            if n_chunks % 2 == 0:
                pltpu.sync_copy(u_hbm.at[src_v.at[n_chunks - 1]], gbuf1)
                pltpu.sync_copy(din_hbm.at[dst_v.at[n_chunks - 1]], dbuf1)
                s_add(n_chunks - 1, gbuf1)
                c_add(n_chunks - 1, dbuf1)
        plsc.subcore_barrier()

        def w_start(k, buf):
            return pltpu.async_copy(
                buf, agg_out.at[cid].at[pl.ds(sid * PT + k * CH, CH)], gsem0)

        def w_wait(buf):
            pltpu.make_async_copy(
                buf, agg_out.at[cid].at[pl.ds(sid * PT, CH)], gsem0).wait()

        pltpu.sync_copy(agg_sh.at[pl.ds(sid * PT, CH)], gbuf0)
        w_start(0, gbuf0)

        def out_pair(i, carry):
            k0 = 2 * i
            pltpu.sync_copy(agg_sh.at[pl.ds(sid * PT + (k0 + 1) * CH, CH)],
                            gbuf1)
            w_start(k0 + 1, gbuf1)
            pltpu.sync_copy(c_sh.at[pl.ds(sid * PT + k0 * CH, CH)], dbuf0)
            pltpu.sync_copy(dbuf0, c_out.at[cid].at[pl.ds(sid * PT + k0 * CH,
                                                          CH)])
            w_wait(gbuf0)
            @pl.when(k0 + 2 < PT // CH)
            def _():
                pltpu.sync_copy(
                    agg_sh.at[pl.ds(sid * PT + (k0 + 2) * CH, CH)], gbuf0)
                w_start(k0 + 2, gbuf0)
            pltpu.sync_copy(c_sh.at[pl.ds(sid * PT + (k0 + 1) * CH, CH)],
                            dbuf0)
            pltpu.sync_copy(dbuf0,
                            c_out.at[cid].at[pl.ds(sid * PT + (k0 + 1) * CH,
                                                   CH)])
            w_wait(gbuf1)
            return carry

        lax.fori_loop(0, PT // CH // 2, out_pair, 0)

    return prop_kernel(src_r, dst_r, u, din16, zeros_tile, zeros16)


def _tc_pre_body(x_ref, w_ref, dpo_ref, dpi_ref, u_ref, din_ref, dout_ref):
    do = (dpo_ref[0] + dpo_ref[1])[:, :1]
    di = (dpi_ref[0] + dpi_ref[1])[:, :1]
    dout = lax.rsqrt(jnp.maximum(do, 1.0))
    din = lax.rsqrt(jnp.maximum(di, 1.0))
    dout_ref[...] = dout
    din_ref[...] = jnp.broadcast_to(din, (NP, W16))
    u_ref[...] = jnp.dot(x_ref[...], w_ref[...],
                         preferred_element_type=jnp.float32) * dout[:N]


def _tc_pre(x, W1, degp_out, degp_in):
    return pl.pallas_call(
        _tc_pre_body,
        out_shape=(
            jax.ShapeDtypeStruct((N, 128), _f32),
            jax.ShapeDtypeStruct((NP, W16), _f32),
            jax.ShapeDtypeStruct((NP, 1), _f32),
        ),
    )(x, W1, degp_out, degp_in)


def _tc_post_body(aggp_ref, cp_ref, din_ref, dout_ref, b1_ref, w2_ref,
                  b2_ref, o_ref):
    agg = aggp_ref[0] + aggp_ref[1]
    din = din_ref[...][:, :1]
    h = jnp.maximum(agg * din + b1_ref[...][None, :], 0.0)
    z = h * dout_ref[...]
    c = (cp_ref[0] + cp_ref[1])[:, :1]
    v = jnp.sum(z * c, axis=0, keepdims=True)
    o_ref[...] = (jnp.dot(v * (1.0 / N), w2_ref[...],
                          preferred_element_type=jnp.float32)
                  + b2_ref[...][None, :])


def _tc_post(aggp, cp, din16, dout_col, b1, W2, b2):
    return pl.pallas_call(
        _tc_post_body,
        out_shape=jax.ShapeDtypeStruct((1, 64), _f32),
    )(aggp, cp, din16, dout_col, b1, W2, b2)


def kernel(x, edge_index, e_h, W1, b1, W2, b2, param_mu, param_sigma):
    del e_h, param_mu, param_sigma
    src_s = edge_index[0].reshape(NW, NCH, CH)
    dst_s = edge_index[1].reshape(NW, NCH, CH)
    ones16 = jnp.ones((CH, W16), _f32)
    zeros16 = jnp.zeros((PT, W16), _f32)
    zeros_tile = jnp.zeros((PT, 128), _f32)

    degp_out, degp_in = _sc_degrees(src_s, dst_s, ones16, zeros16)
    u, din16, dout_col = _tc_pre(x, W1, degp_out, degp_in)
    aggp, cp = _sc_propagate(src_s, dst_s, u, din16, zeros_tile, zeros16)
    return _tc_post(aggp, cp, din16, dout_col, b1, W2, b2)
